# baseline (speedup 1.0000x reference)
import functools

import jax
import jax.numpy as jnp
from jax import lax
from jax.experimental import pallas as pl
from jax.experimental.pallas import tpu as pltpu
from jax.experimental.pallas import tpu_sc as plsc

NUM_FACTORS = 16
RPB = 8
NROWS = 1000000
NBLK = NROWS // RPB
BATCH = 16384
L = 16
NC, NS = 2, 16
NW = NC * NS
BPW = BATCH // NW
CHUNK = 128
NCHUNK = BPW // CHUNK

CT = 48
CROWS = CT * RPB
NCCH = -(-NBLK // CT)
KPW = -(-NCCH // NW)
LAST = NBLK - CT


def _build_convert():
    mesh = plsc.VectorSubcoreMesh(core_axis_name="c", subcore_axis_name="s")

    @functools.partial(
        pl.kernel,
        mesh=mesh,
        compiler_params=pltpu.CompilerParams(needs_layout_passes=False),
        out_type=(
            jax.ShapeDtypeStruct((NBLK, RPB * NUM_FACTORS), jnp.float32),
            jax.ShapeDtypeStruct((NBLK, RPB * NUM_FACTORS), jnp.float32),
        ),
        scratch_types=[
            pltpu.VMEM((CROWS, NUM_FACTORS), jnp.float32),
            pltpu.VMEM((CROWS, NUM_FACTORS), jnp.float32),
            pltpu.VMEM((CT, RPB * NUM_FACTORS), jnp.float32),
            pltpu.SemaphoreType.DMA((2,)),
        ],
    )
    def convert(utab_hbm, etab_hbm, uc_hbm, ec_hbm, pad0, pad1, cf_v, sem):
        wid = lax.axis_index("s") * NC + lax.axis_index("c")

        def blk0(k):
            return jnp.minimum((wid + NW * k) * CT, LAST)

        for tab_hbm, out_hbm in ((utab_hbm, uc_hbm), (etab_hbm, ec_hbm)):
            pltpu.async_copy(
                tab_hbm.at[pl.ds(blk0(0) * RPB, CROWS)], pad0, sem.at[0])

            def step(k, carry):
                def work(pad, b, k=k):
                    nxt = jnp.minimum(k + 1, KPW - 1)
                    other = pad1 if b == 0 else pad0
                    pltpu.async_copy(
                        tab_hbm.at[pl.ds(blk0(nxt) * RPB, CROWS)],
                        other, sem.at[1 - b])
                    pltpu.make_async_copy(
                        tab_hbm.at[pl.ds(0, CROWS)], pad, sem.at[b]).wait()

                    def compact(g, c2):
                        for i in range(L):
                            cf_v[2 * g + i // RPB,
                                 pl.ds((i % RPB) * NUM_FACTORS, NUM_FACTORS)] = (
                                     pad[g * L + i, :])
                        return c2
                    lax.fori_loop(0, CROWS // L, compact, 0)
                    pltpu.sync_copy(cf_v, out_hbm.at[pl.ds(blk0(k), CT)])

                @pl.when(k % 2 == 0)
                def _():
                    work(pad0, 0)

                @pl.when(k % 2 == 1)
                def _():
                    work(pad1, 1)
                return carry

            lax.fori_loop(0, KPW, step, 0)
            lastbuf = pad1 if KPW % 2 == 1 else pad0
            pltpu.make_async_copy(
                tab_hbm.at[pl.ds(0, CROWS)], lastbuf,
                sem.at[KPW % 2]).wait()

    return convert


def _build_gather():
    mesh = plsc.VectorSubcoreMesh(core_axis_name="c", subcore_axis_name="s")

    @functools.partial(
        pl.kernel,
        mesh=mesh,
        compiler_params=pltpu.CompilerParams(needs_layout_passes=False),
        out_type=jax.ShapeDtypeStruct((BATCH,), jnp.float32),
        scratch_types=[
            pltpu.VMEM((NCHUNK, CHUNK), jnp.int32),
            pltpu.VMEM((NCHUNK, CHUNK), jnp.int32),
            pltpu.VMEM((NCHUNK, CHUNK), jnp.int32),
            pltpu.VMEM((NCHUNK, CHUNK), jnp.int32),
            pltpu.VMEM((CHUNK, RPB * NUM_FACTORS), jnp.float32),
            pltpu.VMEM((CHUNK, RPB * NUM_FACTORS), jnp.float32),
            pltpu.VMEM((BPW,), jnp.float32),
            pltpu.SemaphoreType.DMA,
        ],
    )
    def mf_forward(uid_hbm, eid_hbm, uc_hbm, ec_hbm, out_hbm,
                   uid_v, eid_v, ubx_v, ebx_v, u_v, e_v, o_v, sem):
        wid = lax.axis_index("s") * NC + lax.axis_index("c")
        base = wid * BPW
        row0 = wid * NCHUNK

        pltpu.sync_copy(uid_hbm.at[pl.ds(row0, NCHUNK)], uid_v)
        pltpu.sync_copy(eid_hbm.at[pl.ds(row0, NCHUNK)], eid_v)

        for j in range(NCHUNK):
            for o in range(0, CHUNK, L):
                ubx_v[j, pl.ds(o, L)] = jax.lax.shift_right_logical(
                    uid_v[j, pl.ds(o, L)], 3)
                ebx_v[j, pl.ds(o, L)] = jax.lax.shift_right_logical(
                    eid_v[j, pl.ds(o, L)], 3)

        iota = lax.iota(jnp.int32, L)

        for j in range(NCHUNK):
            cu = pltpu.async_copy(uc_hbm.at[ubx_v.at[j]], u_v, sem)
            ce = pltpu.async_copy(ec_hbm.at[ebx_v.at[j]], e_v, sem)
            cu.wait()
            ce.wait()
            for g in range(CHUNK // L):
                rows = g * L + iota
                usub = (uid_v[j, pl.ds(g * L, L)] & 7) * NUM_FACTORS
                esub = (eid_v[j, pl.ds(g * L, L)] & 7) * NUM_FACTORS
                acc = jnp.zeros((L,), jnp.float32)
                for c in range(NUM_FACTORS):
                    u = plsc.load_gather(u_v, [rows, usub + c])
                    e = plsc.load_gather(e_v, [rows, esub + c])
                    acc = acc + u * e
                o_v[pl.ds(j * CHUNK + g * L, L)] = acc

        pltpu.sync_copy(o_v, out_hbm.at[pl.ds(base, BPW)])

    return mf_forward


_CONVERT = _build_convert()
_GATHER = _build_gather()


def kernel(user_id, event_id, user_table, event_table):
    uid2 = user_id.reshape(NW * NCHUNK, CHUNK)
    eid2 = event_id.reshape(NW * NCHUNK, CHUNK)
    uc, ec = _CONVERT(user_table, event_table)
    return _GATHER(uid2, eid2, uc, ec)
